# baseline (speedup 1.0000x reference)
import functools

import jax
import jax.numpy as jnp
from jax import lax
from jax.experimental import pallas as pl
from jax.experimental.pallas import tpu as pltpu
from jax.experimental.pallas import tpu_sc as plsc

EMB = 1024
HEADS = 16
P = 64
BS = 4
N = 4096
NC = 16
R = BS * NC
TOTAL_ROWS = R * P
DH = EMB // HEADS


def _idx_w_body(q_ref, wpts_ref, bpts_ref, ww_ref, bw_ref, gidx_ref, w_ref):
    q = q_ref[...].reshape(R, EMB)
    logits_pts = jnp.dot(q, wpts_ref[...], preferred_element_type=jnp.float32)
    logits_pts = logits_pts + bpts_ref[...]
    idx = jnp.floor(jax.nn.sigmoid(logits_pts) * N).astype(jnp.int32)
    idx = jnp.clip(idx, 0, N - 1)
    row_base = (lax.broadcasted_iota(jnp.int32, (R, P), 0) // NC) * N
    gidx_ref[...] = idx + row_base

    logits_w = jnp.dot(q, ww_ref[...], preferred_element_type=jnp.float32)
    logits_w = logits_w + bw_ref[...]
    m = jnp.max(logits_w, axis=-1, keepdims=True)
    e = jnp.exp(logits_w - m)
    w_ref[...] = e / jnp.sum(e, axis=-1, keepdims=True)


def _idx_w(query, W_pts, b_pts, W_w, b_w):
    return pl.pallas_call(
        _idx_w_body,
        out_shape=(
            jax.ShapeDtypeStruct((R, P), jnp.int32),
            jax.ShapeDtypeStruct((R, P), jnp.float32),
        ),
    )(query, W_pts, b_pts.reshape(1, P), W_w, b_w.reshape(1, P))


_NUM_CORES = 2
_NUM_SUBCORES = 16
_NWORK = _NUM_CORES * _NUM_SUBCORES
_NSPLIT = 2
_GROWS = TOTAL_ROWS // _NSPLIT
_QROWS_PER_W = R // _NSPLIT // _NWORK


def _gather_body(split, table_hbm, idx_hbm, out_hbm, idx_v, rows_v, sem):
    wid = lax.axis_index("s") * _NUM_CORES + lax.axis_index("c")
    for j in range(_QROWS_PER_W):
        row = (split * _NWORK + wid) * _QROWS_PER_W + j
        pltpu.sync_copy(idx_hbm.at[row], idx_v)
        pltpu.async_copy(table_hbm.at[idx_v], rows_v, sem).wait()
        pltpu.sync_copy(
            rows_v,
            out_hbm.at[pl.ds((wid * _QROWS_PER_W + j) * P, P)])


@functools.cache
def _make_gather(split):
    return pl.kernel(
        functools.partial(_gather_body, split),
        out_type=jax.ShapeDtypeStruct((_GROWS, EMB), jnp.float32),
        mesh=plsc.VectorSubcoreMesh(
            core_axis_name="c", subcore_axis_name="s",
            num_cores=_NUM_CORES, num_subcores=_NUM_SUBCORES),
        scratch_types=[
            pltpu.VMEM((P,), jnp.int32),
            pltpu.VMEM((P, EMB), jnp.float32),
            pltpu.SemaphoreType.DMA,
        ],
        name=f"deform_gather_{split}",
    )


_RBLK = 16
_GBLK = _RBLK * P
_GRID = R // _RBLK


_NQ = R // _NSPLIT


def _dense_body(split, g_ref, w_ref, q_ref, wk_ref, bk_ref, wv_ref, bv_ref,
                wp_ref, bp_ref, out_ref):
    f32 = jnp.float32
    bf16 = jnp.bfloat16
    i = pl.program_id(0)
    inv_scale = float(EMB) ** -0.5
    x = g_ref[...].astype(bf16)

    qstep = q_ref[pl.ds(i * _RBLK, _RBLK), :]
    qs16 = (qstep * inv_scale).astype(bf16)
    a_parts = []
    for h in range(HEADS):
        wkh = wk_ref[:, h * DH:(h + 1) * DH]
        qh = qs16[:, h * DH:(h + 1) * DH]
        a_parts.append(lax.dot_general(
            wkh, qh, (((1,), (1,)), ((), ())),
            preferred_element_type=f32).astype(bf16))
    a_sub = jnp.concatenate(a_parts, axis=1)

    ccol = lax.broadcasted_iota(jnp.int32, (_GBLK, _RBLK * HEADS), 1)
    grow = lax.broadcasted_iota(jnp.int32, (_GBLK, _RBLK * HEADS), 0)
    mask = ((ccol % _RBLK) == (grow // P)).astype(f32)
    c2 = lax.broadcasted_iota(jnp.int32, (_RBLK * HEADS, HEADS), 0) // _RBLK
    h2 = lax.broadcasted_iota(jnp.int32, (_RBLK * HEADS, HEADS), 1)
    tsel = (c2 == h2).astype(bf16)
    gg = lax.broadcasted_iota(jnp.int32, (_GBLK, _RBLK), 0) // P
    rr = lax.broadcasted_iota(jnp.int32, (_GBLK, _RBLK), 1)
    rep = (gg == rr).astype(bf16)
    jj = lax.broadcasted_iota(jnp.int32, (EMB, HEADS), 0) // DH
    hh = lax.broadcasted_iota(jnp.int32, (EMB, HEADS), 1)
    m_ind = (jj == hh).astype(bf16)

    cbias = jnp.dot((qstep * (bk_ref[...] * inv_scale)).astype(bf16), m_ind,
                    preferred_element_type=f32)

    e2 = jnp.dot(x, a_sub, preferred_element_type=f32)
    em = (e2 * (mask * w_ref[...])).astype(bf16)
    e_gh = (jnp.dot(em, tsel, preferred_element_type=f32)
            + jnp.dot(rep, cbias.astype(bf16), preferred_element_type=f32))
    ex = jnp.exp(e_gh)
    s = lax.dot_general(rep, ex, (((0,), (0,)), ((), ())),
                        preferred_element_type=f32)

    exm = jnp.dot(ex.astype(bf16), tsel.T, preferred_element_type=f32)
    e2w = (exm * (mask * w_ref[...])).astype(bf16)
    z = lax.dot_general(e2w, x, (((0,), (0,)), ((), ())),
                        preferred_element_type=f32)
    zp = jnp.dot(z.astype(bf16), wv_ref[...],
                 preferred_element_type=f32)

    sinv = 1.0 / s
    mt_f = m_ind.T.astype(f32)
    o = jnp.broadcast_to(bv_ref[...], (_RBLK, EMB))
    for h in range(HEADS):
        o = o + (zp[h * _RBLK:(h + 1) * _RBLK, :]
                 * mt_f[h:h + 1, :] * sinv[:, h:h + 1])
    out_ref[...] = (
        jnp.dot(o.astype(bf16), wp_ref[...], preferred_element_type=f32)
        + bp_ref[...]
    )


def _dense(split, g, w_col, q, wk16, b_k, wv16, b_v, wp16, b_p):
    nrows = R // _NSPLIT
    nsteps = nrows // _RBLK
    off = split * nsteps
    full = lambda shape: pl.BlockSpec(shape, lambda i: (0, 0))
    return pl.pallas_call(
        functools.partial(_dense_body, split),
        grid=(nsteps,),
        in_specs=[
            pl.BlockSpec((_GBLK, EMB), lambda i: (i, 0)),
            pl.BlockSpec((_GBLK, 1), lambda i: (i + off, 0)),
            pl.BlockSpec((_NQ, EMB), lambda i: (split, 0)),
            full((EMB, EMB)),
            full((1, EMB)),
            full((EMB, EMB)),
            full((1, EMB)),
            full((EMB, EMB)),
            full((1, EMB)),
        ],
        out_specs=pl.BlockSpec((_RBLK, EMB), lambda i: (i, 0)),
        out_shape=jax.ShapeDtypeStruct((nrows, EMB), jnp.float32),
        name=f"deform_dense_{split}",
    )(g, w_col, q, wk16, b_k.reshape(1, EMB),
      wv16, b_v.reshape(1, EMB), wp16, b_p.reshape(1, EMB))


def kernel(input, query, W_pts, b_pts, W_w, b_w, W_k, b_k, W_v, b_v, W_p, b_p):
    q = query.reshape(R, EMB)
    gidx, w = _idx_w(query, W_pts, b_pts, W_w, b_w)
    table = input.reshape(BS * N, EMB)
    w_f = w.reshape(TOTAL_ROWS, 1)
    wk16 = W_k.astype(jnp.bfloat16)
    wv16 = W_v.astype(jnp.bfloat16)
    wp16 = W_p.astype(jnp.bfloat16)
    outs = []
    for s in range(_NSPLIT):
        g = _make_gather(s)(table, gidx)
        outs.append(_dense(
            s, g, w_f, q, wk16, b_k, wv16, b_v, wp16, b_p))
    out = jnp.concatenate(outs, axis=0)
    return out.reshape(BS, NC, EMB)


# speedup vs baseline: 1.0519x; 1.0519x over previous
import functools

import jax
import jax.numpy as jnp
from jax import lax
from jax.experimental import pallas as pl
from jax.experimental.pallas import tpu as pltpu
from jax.experimental.pallas import tpu_sc as plsc

EMB = 1024
HEADS = 16
P = 64
BS = 4
N = 4096
NC = 16
R = BS * NC
TOTAL_ROWS = R * P
DH = EMB // HEADS


def _idx_w_body(q_ref, wpts_ref, bpts_ref, ww_ref, bw_ref, gidx_ref, w_ref):
    q = q_ref[...].reshape(R, EMB)
    logits_pts = jnp.dot(q, wpts_ref[...], preferred_element_type=jnp.float32)
    logits_pts = logits_pts + bpts_ref[...]
    idx = jnp.floor(jax.nn.sigmoid(logits_pts) * N).astype(jnp.int32)
    idx = jnp.clip(idx, 0, N - 1)
    row_base = (lax.broadcasted_iota(jnp.int32, (R, P), 0) // NC) * N
    gidx_ref[...] = idx + row_base

    logits_w = jnp.dot(q, ww_ref[...], preferred_element_type=jnp.float32)
    logits_w = logits_w + bw_ref[...]
    m = jnp.max(logits_w, axis=-1, keepdims=True)
    e = jnp.exp(logits_w - m)
    w_ref[...] = e / jnp.sum(e, axis=-1, keepdims=True)


def _idx_w(query, W_pts, b_pts, W_w, b_w):
    return pl.pallas_call(
        _idx_w_body,
        out_shape=(
            jax.ShapeDtypeStruct((R, P), jnp.int32),
            jax.ShapeDtypeStruct((R, P), jnp.float32),
        ),
    )(query, W_pts, b_pts.reshape(1, P), W_w, b_w.reshape(1, P))


_NUM_CORES = 2
_NUM_SUBCORES = 16
_NWORK = _NUM_CORES * _NUM_SUBCORES
_NSPLIT = 1
_GROWS = TOTAL_ROWS // _NSPLIT
_QROWS_PER_W = R // _NSPLIT // _NWORK


def _gather_body(split, table_hbm, idx_hbm, out_hbm, idx_v, rows_v0, rows_v1,
                 sem_g, sem_s):
    wid = lax.axis_index("s") * _NUM_CORES + lax.axis_index("c")
    bufs = [rows_v0, rows_v1]
    nchunk = 2 * _QROWS_PER_W
    half = P // 2
    row0 = (split * _NWORK + wid) * _QROWS_PER_W
    pltpu.sync_copy(idx_hbm.at[pl.ds(row0, _QROWS_PER_W)], idx_v)

    def start_gather(c):
        return pltpu.async_copy(
            table_hbm.at[idx_v.at[c // 2, pl.ds((c % 2) * half, half)]],
            bufs[c % 2], sem_g)

    gathers = [start_gather(c) for c in range(min(2, nchunk))]
    scatters = []
    for c in range(nchunk):
        gathers[c].wait()
        scatters.append(pltpu.async_copy(
            bufs[c % 2],
            out_hbm.at[pl.ds(wid * _QROWS_PER_W * P + c * half, half)],
            sem_s))
        if c + 2 < nchunk:
            scatters[c].wait()
            gathers.append(start_gather(c + 2))
    for c in range(max(0, nchunk - 2), nchunk):
        scatters[c].wait()


@functools.cache
def _make_gather(split):
    return pl.kernel(
        functools.partial(_gather_body, split),
        out_type=jax.ShapeDtypeStruct((_GROWS, EMB), jnp.float32),
        mesh=plsc.VectorSubcoreMesh(
            core_axis_name="c", subcore_axis_name="s",
            num_cores=_NUM_CORES, num_subcores=_NUM_SUBCORES),
        scratch_types=[
            pltpu.VMEM((_QROWS_PER_W, P), jnp.int32),
            pltpu.VMEM((P // 2, EMB), jnp.float32),
            pltpu.VMEM((P // 2, EMB), jnp.float32),
            pltpu.SemaphoreType.DMA,
            pltpu.SemaphoreType.DMA,
        ],
        name=f"deform_gather_{split}",
    )


_RBLK = 16
_GBLK = _RBLK * P
_GRID = R // _RBLK


_NQ = R // _NSPLIT


def _dense_body(split, g_ref, w_ref, q_ref, wk_ref, bk_ref, wv_ref, bv_ref,
                wp_ref, bp_ref, out_ref):
    f32 = jnp.float32
    bf16 = jnp.bfloat16
    i = pl.program_id(0)
    inv_scale = float(EMB) ** -0.5
    x = g_ref[...].astype(bf16)

    qstep = q_ref[pl.ds(i * _RBLK, _RBLK), :]
    qs16 = (qstep * inv_scale).astype(bf16)
    a_parts = []
    for h in range(HEADS):
        wkh = wk_ref[:, h * DH:(h + 1) * DH]
        qh = qs16[:, h * DH:(h + 1) * DH]
        a_parts.append(lax.dot_general(
            wkh, qh, (((1,), (1,)), ((), ())),
            preferred_element_type=f32).astype(bf16))
    a_sub = jnp.concatenate(a_parts, axis=1)

    ccol = lax.broadcasted_iota(jnp.int32, (_GBLK, _RBLK * HEADS), 1)
    grow = lax.broadcasted_iota(jnp.int32, (_GBLK, _RBLK * HEADS), 0)
    mask = ((ccol % _RBLK) == (grow // P)).astype(f32)
    c2 = lax.broadcasted_iota(jnp.int32, (_RBLK * HEADS, HEADS), 0) // _RBLK
    h2 = lax.broadcasted_iota(jnp.int32, (_RBLK * HEADS, HEADS), 1)
    tsel = (c2 == h2).astype(bf16)
    gg = lax.broadcasted_iota(jnp.int32, (_GBLK, _RBLK), 0) // P
    rr = lax.broadcasted_iota(jnp.int32, (_GBLK, _RBLK), 1)
    rep = (gg == rr).astype(bf16)
    jj = lax.broadcasted_iota(jnp.int32, (EMB, HEADS), 0) // DH
    hh = lax.broadcasted_iota(jnp.int32, (EMB, HEADS), 1)
    m_ind = (jj == hh).astype(bf16)

    cbias = jnp.dot((qstep * (bk_ref[...] * inv_scale)).astype(bf16), m_ind,
                    preferred_element_type=f32)

    e2 = jnp.dot(x, a_sub, preferred_element_type=f32)
    em = (e2 * (mask * w_ref[...])).astype(bf16)
    e_gh = (jnp.dot(em, tsel, preferred_element_type=f32)
            + jnp.dot(rep, cbias.astype(bf16), preferred_element_type=f32))
    ex = jnp.exp(e_gh)
    s = lax.dot_general(rep, ex, (((0,), (0,)), ((), ())),
                        preferred_element_type=f32)

    exm = jnp.dot(ex.astype(bf16), tsel.T, preferred_element_type=f32)
    e2w = (exm * (mask * w_ref[...])).astype(bf16)
    z = lax.dot_general(e2w, x, (((0,), (0,)), ((), ())),
                        preferred_element_type=f32)
    zp = jnp.dot(z.astype(bf16), wv_ref[...],
                 preferred_element_type=f32)

    sinv = 1.0 / s
    mt_f = m_ind.T.astype(f32)
    o = jnp.broadcast_to(bv_ref[...], (_RBLK, EMB))
    for h in range(HEADS):
        o = o + (zp[h * _RBLK:(h + 1) * _RBLK, :]
                 * mt_f[h:h + 1, :] * sinv[:, h:h + 1])
    out_ref[...] = (
        jnp.dot(o.astype(bf16), wp_ref[...], preferred_element_type=f32)
        + bp_ref[...]
    )


def _dense(split, g, w_col, q, wk16, b_k, wv16, b_v, wp16, b_p):
    nrows = R // _NSPLIT
    nsteps = nrows // _RBLK
    off = split * nsteps
    full = lambda shape: pl.BlockSpec(shape, lambda i: (0, 0))
    return pl.pallas_call(
        functools.partial(_dense_body, split),
        grid=(nsteps,),
        in_specs=[
            pl.BlockSpec((_GBLK, EMB), lambda i: (i, 0)),
            pl.BlockSpec((_GBLK, 1), lambda i: (i + off, 0)),
            pl.BlockSpec((_NQ, EMB), lambda i: (split, 0)),
            full((EMB, EMB)),
            full((1, EMB)),
            full((EMB, EMB)),
            full((1, EMB)),
            full((EMB, EMB)),
            full((1, EMB)),
        ],
        out_specs=pl.BlockSpec((_RBLK, EMB), lambda i: (i, 0)),
        out_shape=jax.ShapeDtypeStruct((nrows, EMB), jnp.float32),
        name=f"deform_dense_{split}",
    )(g, w_col, q, wk16, b_k.reshape(1, EMB),
      wv16, b_v.reshape(1, EMB), wp16, b_p.reshape(1, EMB))


def kernel(input, query, W_pts, b_pts, W_w, b_w, W_k, b_k, W_v, b_v, W_p, b_p):
    q = query.reshape(R, EMB)
    gidx, w = _idx_w(query, W_pts, b_pts, W_w, b_w)
    table = input.reshape(BS * N, EMB)
    w_f = w.reshape(TOTAL_ROWS, 1)
    wk16 = W_k.astype(jnp.bfloat16)
    wv16 = W_v.astype(jnp.bfloat16)
    wp16 = W_p.astype(jnp.bfloat16)
    outs = []
    for s in range(_NSPLIT):
        g = _make_gather(s)(table, gidx)
        outs.append(_dense(
            s, g, w_f, q, wk16, b_k, wv16, b_v, wp16, b_p))
    out = jnp.concatenate(outs, axis=0)
    return out.reshape(BS, NC, EMB)


# speedup vs baseline: 1.0558x; 1.0038x over previous
import functools

import jax
import jax.numpy as jnp
from jax import lax
from jax.experimental import pallas as pl
from jax.experimental.pallas import tpu as pltpu
from jax.experimental.pallas import tpu_sc as plsc

EMB = 1024
HEADS = 16
P = 64
BS = 4
N = 4096
NC = 16
R = BS * NC
TOTAL_ROWS = R * P
DH = EMB // HEADS


def _idx_w_body(q_ref, wpts_ref, bpts_ref, ww_ref, bw_ref, gidx_ref, w_ref):
    q = q_ref[...].reshape(R, EMB)
    logits_pts = jnp.dot(q, wpts_ref[...], preferred_element_type=jnp.float32)
    logits_pts = logits_pts + bpts_ref[...]
    idx = jnp.floor(jax.nn.sigmoid(logits_pts) * N).astype(jnp.int32)
    idx = jnp.clip(idx, 0, N - 1)
    row_base = (lax.broadcasted_iota(jnp.int32, (R, P), 0) // NC) * N
    gidx_ref[...] = idx + row_base

    logits_w = jnp.dot(q, ww_ref[...], preferred_element_type=jnp.float32)
    logits_w = logits_w + bw_ref[...]
    m = jnp.max(logits_w, axis=-1, keepdims=True)
    e = jnp.exp(logits_w - m)
    w_ref[...] = e / jnp.sum(e, axis=-1, keepdims=True)


def _idx_w(query, W_pts, b_pts, W_w, b_w):
    return pl.pallas_call(
        _idx_w_body,
        out_shape=(
            jax.ShapeDtypeStruct((R, P), jnp.int32),
            jax.ShapeDtypeStruct((R, P), jnp.float32),
        ),
    )(query, W_pts, b_pts.reshape(1, P), W_w, b_w.reshape(1, P))


_NUM_CORES = 2
_NUM_SUBCORES = 16
_NWORK = _NUM_CORES * _NUM_SUBCORES
_NSPLIT = 1
_GROWS = TOTAL_ROWS // _NSPLIT
_QROWS_PER_W = R // _NSPLIT // _NWORK


def _gather_body(split, table_hbm, idx_hbm, out_hbm, idx_v, rows_v, sem):
    wid = lax.axis_index("s") * _NUM_CORES + lax.axis_index("c")
    for j in range(_QROWS_PER_W):
        row = (split * _NWORK + wid) * _QROWS_PER_W + j
        pltpu.sync_copy(idx_hbm.at[row], idx_v)
        pltpu.async_copy(table_hbm.at[idx_v], rows_v, sem).wait()
        pltpu.sync_copy(
            rows_v,
            out_hbm.at[pl.ds((wid * _QROWS_PER_W + j) * P, P)])


@functools.cache
def _make_gather(split):
    return pl.kernel(
        functools.partial(_gather_body, split),
        out_type=jax.ShapeDtypeStruct((_GROWS, EMB), jnp.float32),
        mesh=plsc.VectorSubcoreMesh(
            core_axis_name="c", subcore_axis_name="s",
            num_cores=_NUM_CORES, num_subcores=_NUM_SUBCORES),
        scratch_types=[
            pltpu.VMEM((P,), jnp.int32),
            pltpu.VMEM((P, EMB), jnp.float32),
            pltpu.SemaphoreType.DMA,
        ],
        name=f"deform_gather_{split}",
    )


_RBLK = 16
_GBLK = _RBLK * P
_GRID = R // _RBLK


_NQ = R // _NSPLIT


def _dense_body(split, g_ref, w_ref, q_ref, wk_ref, bk_ref, wv_ref, bv_ref,
                wp_ref, bp_ref, out_ref):
    f32 = jnp.float32
    bf16 = jnp.bfloat16
    i = pl.program_id(0)
    inv_scale = float(EMB) ** -0.5
    x = g_ref[...].astype(bf16)

    qstep = q_ref[pl.ds(i * _RBLK, _RBLK), :]
    qs16 = (qstep * inv_scale).astype(bf16)
    a_parts = []
    for h in range(HEADS):
        wkh = wk_ref[:, h * DH:(h + 1) * DH]
        qh = qs16[:, h * DH:(h + 1) * DH]
        a_parts.append(lax.dot_general(
            wkh, qh, (((1,), (1,)), ((), ())),
            preferred_element_type=f32).astype(bf16))
    a_sub = jnp.concatenate(a_parts, axis=1)

    ccol = lax.broadcasted_iota(jnp.int32, (_GBLK, _RBLK * HEADS), 1)
    grow = lax.broadcasted_iota(jnp.int32, (_GBLK, _RBLK * HEADS), 0)
    mask = ((ccol % _RBLK) == (grow // P)).astype(f32)
    c2 = lax.broadcasted_iota(jnp.int32, (_RBLK * HEADS, HEADS), 0) // _RBLK
    h2 = lax.broadcasted_iota(jnp.int32, (_RBLK * HEADS, HEADS), 1)
    tsel = (c2 == h2).astype(bf16)
    gg = lax.broadcasted_iota(jnp.int32, (_GBLK, _RBLK), 0) // P
    rr = lax.broadcasted_iota(jnp.int32, (_GBLK, _RBLK), 1)
    rep = (gg == rr).astype(bf16)
    jj = lax.broadcasted_iota(jnp.int32, (EMB, HEADS), 0) // DH
    hh = lax.broadcasted_iota(jnp.int32, (EMB, HEADS), 1)
    m_ind = (jj == hh).astype(bf16)

    cbias = jnp.dot((qstep * (bk_ref[...] * inv_scale)).astype(bf16), m_ind,
                    preferred_element_type=f32)

    e2 = jnp.dot(x, a_sub, preferred_element_type=f32)
    em = (e2 * (mask * w_ref[...])).astype(bf16)
    e_gh = (jnp.dot(em, tsel, preferred_element_type=f32)
            + jnp.dot(rep, cbias.astype(bf16), preferred_element_type=f32))
    ex = jnp.exp(e_gh)
    s = lax.dot_general(rep, ex, (((0,), (0,)), ((), ())),
                        preferred_element_type=f32)

    exm = jnp.dot(ex.astype(bf16), tsel.T, preferred_element_type=f32)
    e2w = (exm * (mask * w_ref[...])).astype(bf16)
    z = lax.dot_general(e2w, x, (((0,), (0,)), ((), ())),
                        preferred_element_type=f32)
    zp = jnp.dot(z.astype(bf16), wv_ref[...],
                 preferred_element_type=f32)

    sinv = 1.0 / s
    mt_f = m_ind.T.astype(f32)
    o = jnp.broadcast_to(bv_ref[...], (_RBLK, EMB))
    for h in range(HEADS):
        o = o + (zp[h * _RBLK:(h + 1) * _RBLK, :]
                 * mt_f[h:h + 1, :] * sinv[:, h:h + 1])
    out_ref[...] = (
        jnp.dot(o.astype(bf16), wp_ref[...], preferred_element_type=f32)
        + bp_ref[...]
    )


def _dense(split, g, w_col, q, wk16, b_k, wv16, b_v, wp16, b_p):
    nrows = R // _NSPLIT
    nsteps = nrows // _RBLK
    off = split * nsteps
    full = lambda shape: pl.BlockSpec(shape, lambda i: (0, 0))
    return pl.pallas_call(
        functools.partial(_dense_body, split),
        grid=(nsteps,),
        in_specs=[
            pl.BlockSpec((_GBLK, EMB), lambda i: (i, 0)),
            pl.BlockSpec((_GBLK, 1), lambda i: (i + off, 0)),
            pl.BlockSpec((_NQ, EMB), lambda i: (split, 0)),
            full((EMB, EMB)),
            full((1, EMB)),
            full((EMB, EMB)),
            full((1, EMB)),
            full((EMB, EMB)),
            full((1, EMB)),
        ],
        out_specs=pl.BlockSpec((_RBLK, EMB), lambda i: (i, 0)),
        out_shape=jax.ShapeDtypeStruct((nrows, EMB), jnp.float32),
        name=f"deform_dense_{split}",
    )(g, w_col, q, wk16, b_k.reshape(1, EMB),
      wv16, b_v.reshape(1, EMB), wp16, b_p.reshape(1, EMB))


def kernel(input, query, W_pts, b_pts, W_w, b_w, W_k, b_k, W_v, b_v, W_p, b_p):
    q = query.reshape(R, EMB)
    gidx, w = _idx_w(query, W_pts, b_pts, W_w, b_w)
    table = input.reshape(BS * N, EMB)
    w_f = w.reshape(TOTAL_ROWS, 1)
    wk16 = W_k.astype(jnp.bfloat16)
    wv16 = W_v.astype(jnp.bfloat16)
    wp16 = W_p.astype(jnp.bfloat16)
    outs = []
    for s in range(_NSPLIT):
        g = _make_gather(s)(table, gidx)
        outs.append(_dense(
            s, g, w_f, q, wk16, b_k, wv16, b_v, wp16, b_p))
    out = jnp.concatenate(outs, axis=0)
    return out.reshape(BS, NC, EMB)
